# Initial kernel scaffold; baseline (speedup 1.0000x reference)
import jax
import jax.numpy as jnp
from jax.experimental import pallas as pl


def kernel(x, edge_index):
    raise NotImplementedError("write your pallas kernel here")


# speedup vs baseline: 2.1209x; 2.1209x over previous
import functools

import jax
import jax.numpy as jnp
from jax import lax
from jax.experimental import pallas as pl
from jax.experimental.pallas import tpu as pltpu
from jax.experimental.pallas import tpu_sc as plsc

E = 320000
D = 128
N_CORES = 2
N_SUBCORES = 16
NW = N_CORES * N_SUBCORES
EPW = E // NW
C = 80
NCHUNK = EPW // C
G = C // 16
KD = D // 16


def _dot_body(x_hbm, src_hbm, dst_hbm, out_hbm,
              src_v, dst_v, u_v, v_v, o_v, sem_u, sem_v):
    wid = lax.axis_index("s") * N_CORES + lax.axis_index("c")
    iota = lax.iota(jnp.int32, 16)

    def chunk_body(ci, carry):
        base = wid * EPW + ci * C
        pltpu.sync_copy(src_hbm.at[pl.ds(base, C)], src_v)
        pltpu.sync_copy(dst_hbm.at[pl.ds(base, C)], dst_v)
        cp_u = pltpu.async_copy(x_hbm.at[src_v], u_v, sem_u)
        cp_v = pltpu.async_copy(x_hbm.at[dst_v], v_v, sem_v)
        cp_u.wait()
        cp_v.wait()

        for g in range(G):
            sv = jnp.zeros((16,), jnp.float32)
            for j in range(16):
                e = g * 16 + j
                acc = u_v[e, pl.ds(0, 16)] * v_v[e, pl.ds(0, 16)]
                for k in range(1, KD):
                    acc = acc + u_v[e, pl.ds(k * 16, 16)] * v_v[e, pl.ds(k * 16, 16)]
                s = jnp.sum(acc)
                sv = jnp.where(iota == j, s, sv)
            o_v[pl.ds(g * 16, 16)] = sv
        pltpu.sync_copy(o_v, out_hbm.at[pl.ds(base, C)])
        return carry

    lax.fori_loop(0, NCHUNK, chunk_body, 0)


@jax.jit
def kernel(x, edge_index):
    src = edge_index[0]
    dst = edge_index[1]
    mesh = plsc.VectorSubcoreMesh(
        core_axis_name="c", subcore_axis_name="s",
        num_cores=N_CORES, num_subcores=N_SUBCORES)
    f = pl.kernel(
        _dot_body,
        out_type=jax.ShapeDtypeStruct((E,), jnp.float32),
        mesh=mesh,
        compiler_params=pltpu.CompilerParams(needs_layout_passes=False),
        scratch_types=[
            pltpu.VMEM((C,), jnp.int32),
            pltpu.VMEM((C,), jnp.int32),
            pltpu.VMEM((C, D), jnp.float32),
            pltpu.VMEM((C, D), jnp.float32),
            pltpu.VMEM((C,), jnp.float32),
            pltpu.SemaphoreType.DMA,
            pltpu.SemaphoreType.DMA,
        ],
    )
    return f(x, src, dst)


# speedup vs baseline: 12.6135x; 5.9471x over previous
import jax
import jax.numpy as jnp
from jax import lax
from jax.experimental import pallas as pl
from jax.experimental.pallas import tpu as pltpu
from jax.experimental.pallas import tpu_sc as plsc

E = 320000
D = 128
W = D // 2
N_CORES = 2
N_SUBCORES = 16
NW = N_CORES * N_SUBCORES
EPW = E // NW
C = 80
NCHUNK = EPW // C
G = C // 16
KW = W // 16
NB = 5
MASK = -65536


def _dot_body(x_hbm, src_hbm, dst_hbm, out_hbm,
              srcs, dsts, u_v, v_v, o_v, sems, osems):
    wid = lax.axis_index("s") * N_CORES + lax.axis_index("c")
    iota = lax.iota(jnp.int32, 16)
    ebase = wid * EPW

    pltpu.sync_copy(src_hbm.at[pl.ds(ebase, EPW)], srcs)
    pltpu.sync_copy(dst_hbm.at[pl.ds(ebase, EPW)], dsts)

    def issue(ci, b):
        pltpu.async_copy(x_hbm.at[srcs.at[pl.ds(ci * C, C)]], u_v.at[b], sems.at[b])
        pltpu.async_copy(x_hbm.at[dsts.at[pl.ds(ci * C, C)]], v_v.at[b], sems.at[b])

    for b in range(NB):
        issue(b, b)

    def chunk_body(ci, carry):
        b = lax.rem(ci, NB)

        @pl.when(ci >= NB)
        def _():
            pltpu.make_async_copy(
                o_v.at[b], out_hbm.at[pl.ds(0, C)], osems.at[b]).wait()

        pltpu.make_async_copy(
            x_hbm.at[srcs.at[pl.ds(ci * C, C)]], u_v.at[b], sems.at[b]).wait()
        pltpu.make_async_copy(
            x_hbm.at[dsts.at[pl.ds(ci * C, C)]], v_v.at[b], sems.at[b]).wait()

        for g in range(G):
            sv = jnp.zeros((16,), jnp.float32)
            for j in range(16):
                e = g * 16 + j
                acc = None
                for k in range(KW):
                    wu = u_v[b, e, pl.ds(k * 16, 16)]
                    wv = v_v[b, e, pl.ds(k * 16, 16)]
                    if k % 2 == 0:
                        ul = plsc.bitcast(wu << 16, jnp.float32)
                        vl = plsc.bitcast(wv << 16, jnp.float32)
                        uh = plsc.bitcast(wu & MASK, jnp.float32)
                        vh = plsc.bitcast(wv & MASK, jnp.float32)
                        t = ul * vl + uh * vh
                    else:
                        bu = plsc.bitcast(wu, jnp.bfloat16)
                        bv = plsc.bitcast(wv, jnp.bfloat16)
                        pr = bu * bv
                        p0, p1 = plsc.unpack(pr, format=plsc.PackFormat.INTERLEAVED)
                        t = p0 + p1
                    acc = t if acc is None else acc + t
                s = jnp.sum(acc)
                sv = jnp.where(iota == j, s, sv)
            o_v[b, pl.ds(g * 16, 16)] = sv

        pltpu.async_copy(o_v.at[b], out_hbm.at[pl.ds(ebase + ci * C, C)], osems.at[b])

        @pl.when(ci + NB < NCHUNK)
        def _():
            issue_ci = ci + NB
            pltpu.async_copy(
                x_hbm.at[srcs.at[pl.ds(issue_ci * C, C)]], u_v.at[b], sems.at[b])
            pltpu.async_copy(
                x_hbm.at[dsts.at[pl.ds(issue_ci * C, C)]], v_v.at[b], sems.at[b])
        return carry

    lax.fori_loop(0, NCHUNK, chunk_body, 0)

    for b in range(NB):
        pltpu.make_async_copy(
            o_v.at[b], out_hbm.at[pl.ds(0, C)], osems.at[b]).wait()


@jax.jit
def kernel(x, edge_index):
    src = edge_index[0]
    dst = edge_index[1]
    xi = lax.bitcast_convert_type(
        x.astype(jnp.bfloat16).reshape(x.shape[0], W, 2), jnp.int32)
    mesh = plsc.VectorSubcoreMesh(
        core_axis_name="c", subcore_axis_name="s",
        num_cores=N_CORES, num_subcores=N_SUBCORES)
    f = pl.kernel(
        _dot_body,
        out_type=jax.ShapeDtypeStruct((E,), jnp.float32),
        mesh=mesh,
        compiler_params=pltpu.CompilerParams(needs_layout_passes=False, use_tc_tiling_on_sc=False),
        scratch_types=[
            pltpu.VMEM((EPW,), jnp.int32),
            pltpu.VMEM((EPW,), jnp.int32),
            pltpu.VMEM((NB, C, W), jnp.int32),
            pltpu.VMEM((NB, C, W), jnp.int32),
            pltpu.VMEM((NB, C), jnp.float32),
            pltpu.SemaphoreType.DMA((NB,)),
            pltpu.SemaphoreType.DMA((NB,)),
        ],
    )
    return f(xi, src, dst)


# speedup vs baseline: 15.0741x; 1.1951x over previous
import jax
import jax.numpy as jnp
from jax import lax
from jax.experimental import pallas as pl
from jax.experimental.pallas import tpu as pltpu
from jax.experimental.pallas import tpu_sc as plsc

E = 320000
N = 10000
D = 128
W = D // 2
N_CORES = 2
N_SUBCORES = 16
NW = N_CORES * N_SUBCORES
EPW = E // NW
C = 80
NCHUNK = EPW // C
G = C // 16
KW = W // 16
NB = 4
RPT = N // N_SUBCORES
PR = 25
NP1 = RPT // PR
PNB = 3


def _dot_body(x_hbm, src_hbm, dst_hbm, out_hbm, packed_hbm,
              srcs, dsts, xf_v, pk_v, u_v, v_v, o_v,
              sems, osems, pisems, posems):
    cid = lax.axis_index("c")
    sid = lax.axis_index("s")
    wid = sid * N_CORES + cid
    iota = lax.iota(jnp.int32, 16)
    ebase = wid * EPW
    row0 = sid * RPT
    pbase = cid * N

    def p1_issue(i, b):
        pltpu.async_copy(
            x_hbm.at[pl.ds(row0 + i * PR, PR)], xf_v.at[b], pisems.at[b])

    for b in range(PNB):
        p1_issue(b, b)

    def p1_body(i, carry):
        b = lax.rem(i, PNB)

        @pl.when(i >= PNB)
        def _():
            pltpu.make_async_copy(
                pk_v.at[b], packed_hbm.at[pl.ds(0, PR)], posems.at[b]).wait()

        pltpu.make_async_copy(
            x_hbm.at[pl.ds(0, PR)], xf_v.at[b], pisems.at[b]).wait()

        for r in range(PR):
            for k in range(KW):
                a = xf_v[b, r, pl.ds(k * 32, 16)]
                c2 = xf_v[b, r, pl.ds(k * 32 + 16, 16)]
                w = plsc.pack(a, c2, format=plsc.PackFormat.INTERLEAVED)
                pk_v[b, r, pl.ds(k * 16, 16)] = plsc.bitcast(w, jnp.int32)

        pltpu.async_copy(
            pk_v.at[b],
            packed_hbm.at[pl.ds(pbase + row0 + i * PR, PR)], posems.at[b])

        @pl.when(i + PNB < NP1)
        def _():
            p1_issue(i + PNB, b)
        return carry

    lax.fori_loop(0, NP1, p1_body, 0)

    for b in range(PNB):
        pltpu.make_async_copy(
            pk_v.at[b], packed_hbm.at[pl.ds(0, PR)], posems.at[b]).wait()

    plsc.subcore_barrier()

    pview = packed_hbm.at[pl.ds(pbase, N)]

    pltpu.sync_copy(src_hbm.at[pl.ds(ebase, EPW)], srcs)
    pltpu.sync_copy(dst_hbm.at[pl.ds(ebase, EPW)], dsts)

    def issue(ci, b):
        pltpu.async_copy(pview.at[srcs.at[pl.ds(ci * C, C)]], u_v.at[b], sems.at[b])
        pltpu.async_copy(pview.at[dsts.at[pl.ds(ci * C, C)]], v_v.at[b], sems.at[b])

    for b in range(NB):
        issue(b, b)

    def chunk_body(ci, carry):
        b = lax.rem(ci, NB)

        @pl.when(ci >= NB)
        def _():
            pltpu.make_async_copy(
                o_v.at[b], out_hbm.at[pl.ds(0, C)], osems.at[b]).wait()

        pltpu.make_async_copy(
            pview.at[srcs.at[pl.ds(ci * C, C)]], u_v.at[b], sems.at[b]).wait()
        pltpu.make_async_copy(
            pview.at[dsts.at[pl.ds(ci * C, C)]], v_v.at[b], sems.at[b]).wait()

        for g in range(G):
            sv = jnp.zeros((16,), jnp.float32)
            for j in range(16):
                e = g * 16 + j
                acc = None
                for k in range(KW):
                    wu = u_v[b, e, pl.ds(k * 16, 16)]
                    wv = v_v[b, e, pl.ds(k * 16, 16)]
                    bu = plsc.bitcast(wu, jnp.bfloat16)
                    bv = plsc.bitcast(wv, jnp.bfloat16)
                    pr = bu * bv
                    p0, p1 = plsc.unpack(pr, format=plsc.PackFormat.INTERLEAVED)
                    t = p0 + p1
                    acc = t if acc is None else acc + t
                s = jnp.sum(acc)
                sv = jnp.where(iota == j, s, sv)
            o_v[b, pl.ds(g * 16, 16)] = sv

        pltpu.async_copy(o_v.at[b], out_hbm.at[pl.ds(ebase + ci * C, C)], osems.at[b])

        @pl.when(ci + NB < NCHUNK)
        def _():
            issue_ci = ci + NB
            pltpu.async_copy(
                pview.at[srcs.at[pl.ds(issue_ci * C, C)]], u_v.at[b], sems.at[b])
            pltpu.async_copy(
                pview.at[dsts.at[pl.ds(issue_ci * C, C)]], v_v.at[b], sems.at[b])
        return carry

    lax.fori_loop(0, NCHUNK, chunk_body, 0)

    for b in range(NB):
        pltpu.make_async_copy(
            o_v.at[b], out_hbm.at[pl.ds(0, C)], osems.at[b]).wait()


@jax.jit
def kernel(x, edge_index):
    src = edge_index[0]
    dst = edge_index[1]
    mesh = plsc.VectorSubcoreMesh(
        core_axis_name="c", subcore_axis_name="s",
        num_cores=N_CORES, num_subcores=N_SUBCORES)
    f = pl.kernel(
        _dot_body,
        out_type=[
            jax.ShapeDtypeStruct((E,), jnp.float32),
            jax.ShapeDtypeStruct((N_CORES * N, W), jnp.int32),
        ],
        mesh=mesh,
        compiler_params=pltpu.CompilerParams(needs_layout_passes=False, use_tc_tiling_on_sc=False),
        scratch_types=[
            pltpu.VMEM((EPW,), jnp.int32),
            pltpu.VMEM((EPW,), jnp.int32),
            pltpu.VMEM((PNB, PR, D), jnp.float32),
            pltpu.VMEM((PNB, PR, W), jnp.int32),
            pltpu.VMEM((NB, C, W), jnp.int32),
            pltpu.VMEM((NB, C, W), jnp.int32),
            pltpu.VMEM((NB, C), jnp.float32),
            pltpu.SemaphoreType.DMA((NB,)),
            pltpu.SemaphoreType.DMA((NB,)),
            pltpu.SemaphoreType.DMA((PNB,)),
            pltpu.SemaphoreType.DMA((PNB,)),
        ],
    )
    scores, _ = f(x, src, dst)
    return scores


# speedup vs baseline: 16.5788x; 1.0998x over previous
import jax
import jax.numpy as jnp
from jax import lax
from jax.experimental import pallas as pl
from jax.experimental.pallas import tpu as pltpu
from jax.experimental.pallas import tpu_sc as plsc

E = 320000
N = 10000
D = 128
W = D // 2
N_CORES = 2
N_SUBCORES = 16
NW = N_CORES * N_SUBCORES
EPW = E // NW
C = 80
NCHUNK = EPW // C
G = C // 16
KW = W // 16
NB = 4
RPT = N // N_SUBCORES
PR = 25
NP1 = RPT // PR
PNB = 3


def _dot_body(x_hbm, ei_hbm, out_hbm, packed_hbm,
              srcs, dsts, xf_v, pk_v, u_v, v_v, o_v,
              sems, osems, pisems, posems):
    cid = lax.axis_index("c")
    sid = lax.axis_index("s")
    wid = sid * N_CORES + cid
    iota = lax.iota(jnp.int32, 16)
    ebase = wid * EPW
    row0 = sid * RPT
    pbase = cid * N

    def p1_issue(i, b):
        pltpu.async_copy(
            x_hbm.at[pl.ds(row0 + i * PR, PR)], xf_v.at[b], pisems.at[b])

    for b in range(PNB):
        p1_issue(b, b)

    def p1_body(i, carry):
        b = lax.rem(i, PNB)

        @pl.when(i >= PNB)
        def _():
            pltpu.make_async_copy(
                pk_v.at[b], packed_hbm.at[pl.ds(0, PR)], posems.at[b]).wait()

        pltpu.make_async_copy(
            x_hbm.at[pl.ds(0, PR)], xf_v.at[b], pisems.at[b]).wait()

        for r in range(PR):
            for k in range(KW):
                a = xf_v[b, r, pl.ds(k * 32, 16)]
                c2 = xf_v[b, r, pl.ds(k * 32 + 16, 16)]
                w = plsc.pack(a, c2, format=plsc.PackFormat.INTERLEAVED)
                pk_v[b, r, pl.ds(k * 16, 16)] = plsc.bitcast(w, jnp.int32)

        pltpu.async_copy(
            pk_v.at[b],
            packed_hbm.at[pl.ds(pbase + row0 + i * PR, PR)], posems.at[b])

        @pl.when(i + PNB < NP1)
        def _():
            p1_issue(i + PNB, b)
        return carry

    lax.fori_loop(0, NP1, p1_body, 0)

    for b in range(PNB):
        pltpu.make_async_copy(
            pk_v.at[b], packed_hbm.at[pl.ds(0, PR)], posems.at[b]).wait()

    plsc.subcore_barrier()

    pview = packed_hbm.at[pl.ds(pbase, N)]

    pltpu.sync_copy(ei_hbm.at[0, pl.ds(ebase, EPW)], srcs)
    pltpu.sync_copy(ei_hbm.at[1, pl.ds(ebase, EPW)], dsts)

    def issue(ci, b):
        pltpu.async_copy(pview.at[srcs.at[pl.ds(ci * C, C)]], u_v.at[b], sems.at[b])
        pltpu.async_copy(pview.at[dsts.at[pl.ds(ci * C, C)]], v_v.at[b], sems.at[b])

    for b in range(NB):
        issue(b, b)

    def chunk_body(ci, carry):
        b = lax.rem(ci, NB)

        @pl.when(ci >= NB)
        def _():
            pltpu.make_async_copy(
                o_v.at[b], out_hbm.at[pl.ds(0, C)], osems.at[b]).wait()

        pltpu.make_async_copy(
            pview.at[srcs.at[pl.ds(ci * C, C)]], u_v.at[b], sems.at[b]).wait()
        pltpu.make_async_copy(
            pview.at[dsts.at[pl.ds(ci * C, C)]], v_v.at[b], sems.at[b]).wait()

        for g in range(G):
            sv = jnp.zeros((16,), jnp.float32)
            for j in range(16):
                e = g * 16 + j
                acc = None
                for k in range(KW):
                    wu = u_v[b, e, pl.ds(k * 16, 16)]
                    wv = v_v[b, e, pl.ds(k * 16, 16)]
                    bu = plsc.bitcast(wu, jnp.bfloat16)
                    bv = plsc.bitcast(wv, jnp.bfloat16)
                    pr = bu * bv
                    p0, p1 = plsc.unpack(pr, format=plsc.PackFormat.INTERLEAVED)
                    t = p0 + p1
                    acc = t if acc is None else acc + t
                s = jnp.sum(acc)
                sv = jnp.where(iota == j, s, sv)
            o_v[b, pl.ds(g * 16, 16)] = sv

        pltpu.async_copy(o_v.at[b], out_hbm.at[pl.ds(ebase + ci * C, C)], osems.at[b])

        @pl.when(ci + NB < NCHUNK)
        def _():
            issue_ci = ci + NB
            pltpu.async_copy(
                pview.at[srcs.at[pl.ds(issue_ci * C, C)]], u_v.at[b], sems.at[b])
            pltpu.async_copy(
                pview.at[dsts.at[pl.ds(issue_ci * C, C)]], v_v.at[b], sems.at[b])
        return carry

    lax.fori_loop(0, NCHUNK, chunk_body, 0)

    for b in range(NB):
        pltpu.make_async_copy(
            o_v.at[b], out_hbm.at[pl.ds(0, C)], osems.at[b]).wait()


@jax.jit
def kernel(x, edge_index):
    mesh = plsc.VectorSubcoreMesh(
        core_axis_name="c", subcore_axis_name="s",
        num_cores=N_CORES, num_subcores=N_SUBCORES)
    f = pl.kernel(
        _dot_body,
        out_type=[
            jax.ShapeDtypeStruct((E,), jnp.float32),
            jax.ShapeDtypeStruct((N_CORES * N, W), jnp.int32),
        ],
        mesh=mesh,
        compiler_params=pltpu.CompilerParams(needs_layout_passes=False, use_tc_tiling_on_sc=False),
        scratch_types=[
            pltpu.VMEM((EPW,), jnp.int32),
            pltpu.VMEM((EPW,), jnp.int32),
            pltpu.VMEM((PNB, PR, D), jnp.float32),
            pltpu.VMEM((PNB, PR, W), jnp.int32),
            pltpu.VMEM((NB, C, W), jnp.int32),
            pltpu.VMEM((NB, C, W), jnp.int32),
            pltpu.VMEM((NB, C), jnp.float32),
            pltpu.SemaphoreType.DMA((NB,)),
            pltpu.SemaphoreType.DMA((NB,)),
            pltpu.SemaphoreType.DMA((PNB,)),
            pltpu.SemaphoreType.DMA((PNB,)),
        ],
    )
    scores, _ = f(x, edge_index)
    return scores


# speedup vs baseline: 16.8975x; 1.0192x over previous
import jax
import jax.numpy as jnp
from jax import lax
from jax.experimental import pallas as pl
from jax.experimental.pallas import tpu as pltpu
from jax.experimental.pallas import tpu_sc as plsc

E = 320000
N = 10000
D = 128
W = D // 2
N_CORES = 2
N_SUBCORES = 16
NW = N_CORES * N_SUBCORES
EPW = E // NW
C = 80
NCHUNK = EPW // C
G = C // 16
KW = W // 16
NB = 4
RPT = N // N_SUBCORES
PR = 25
NP1 = RPT // PR
PNB = 3


def _dot_body(x_hbm, ei_hbm, out_hbm, packed_hbm,
              srcs, dsts, xf_v, pk_v, u_v, v_v, o_v,
              sems, osems, pisems, posems, isems):
    cid = lax.axis_index("c")
    sid = lax.axis_index("s")
    wid = sid * N_CORES + cid
    iota = lax.iota(jnp.int32, 16)
    ebase = wid * EPW
    row0 = sid * RPT
    pbase = cid * N

    pltpu.async_copy(ei_hbm.at[0, pl.ds(ebase, EPW)], srcs, isems.at[0])
    pltpu.async_copy(ei_hbm.at[1, pl.ds(ebase, EPW)], dsts, isems.at[1])

    def p1_issue(i, b):
        pltpu.async_copy(
            x_hbm.at[pl.ds(row0 + i * PR, PR)], xf_v.at[b], pisems.at[b])

    for b in range(PNB):
        p1_issue(b, b)

    def p1_body(i, carry):
        b = lax.rem(i, PNB)

        @pl.when(i >= PNB)
        def _():
            pltpu.make_async_copy(
                pk_v.at[b], packed_hbm.at[pl.ds(0, PR)], posems.at[b]).wait()

        pltpu.make_async_copy(
            x_hbm.at[pl.ds(0, PR)], xf_v.at[b], pisems.at[b]).wait()

        for r in range(PR):
            for k in range(KW):
                a = xf_v[b, r, pl.ds(k * 32, 16)]
                c2 = xf_v[b, r, pl.ds(k * 32 + 16, 16)]
                w = plsc.pack(a, c2, format=plsc.PackFormat.INTERLEAVED)
                pk_v[b, r, pl.ds(k * 16, 16)] = plsc.bitcast(w, jnp.int32)

        pltpu.async_copy(
            pk_v.at[b],
            packed_hbm.at[pl.ds(pbase + row0 + i * PR, PR)], posems.at[b])

        @pl.when(i + PNB < NP1)
        def _():
            p1_issue(i + PNB, b)
        return carry

    lax.fori_loop(0, NP1, p1_body, 0)

    for b in range(PNB):
        pltpu.make_async_copy(
            pk_v.at[b], packed_hbm.at[pl.ds(0, PR)], posems.at[b]).wait()

    plsc.subcore_barrier()

    pview = packed_hbm.at[pl.ds(pbase, N)]

    pltpu.make_async_copy(ei_hbm.at[0, pl.ds(ebase, EPW)], srcs, isems.at[0]).wait()
    pltpu.make_async_copy(ei_hbm.at[1, pl.ds(ebase, EPW)], dsts, isems.at[1]).wait()

    def issue(ci, b):
        pltpu.async_copy(pview.at[srcs.at[pl.ds(ci * C, C)]], u_v.at[b], sems.at[b])
        pltpu.async_copy(pview.at[dsts.at[pl.ds(ci * C, C)]], v_v.at[b], sems.at[b])

    for b in range(NB):
        issue(b, b)

    def chunk_body(ci, carry):
        b = lax.rem(ci, NB)

        @pl.when(ci >= NB)
        def _():
            pltpu.make_async_copy(
                o_v.at[b], out_hbm.at[pl.ds(0, C)], osems.at[b]).wait()

        pltpu.make_async_copy(
            pview.at[srcs.at[pl.ds(ci * C, C)]], u_v.at[b], sems.at[b]).wait()
        pltpu.make_async_copy(
            pview.at[dsts.at[pl.ds(ci * C, C)]], v_v.at[b], sems.at[b]).wait()

        for g in range(G):
            sv = jnp.zeros((16,), jnp.float32)
            for j in range(16):
                e = g * 16 + j
                acc = None
                for k in range(KW):
                    wu = u_v[b, e, pl.ds(k * 16, 16)]
                    wv = v_v[b, e, pl.ds(k * 16, 16)]
                    bu = plsc.bitcast(wu, jnp.bfloat16)
                    bv = plsc.bitcast(wv, jnp.bfloat16)
                    pr = bu * bv
                    p0, p1 = plsc.unpack(pr, format=plsc.PackFormat.INTERLEAVED)
                    t = p0 + p1
                    acc = t if acc is None else acc + t
                s = jnp.sum(acc)
                sv = jnp.where(iota == j, s, sv)
            o_v[b, pl.ds(g * 16, 16)] = sv

        pltpu.async_copy(o_v.at[b], out_hbm.at[pl.ds(ebase + ci * C, C)], osems.at[b])

        @pl.when(ci + NB < NCHUNK)
        def _():
            issue_ci = ci + NB
            pltpu.async_copy(
                pview.at[srcs.at[pl.ds(issue_ci * C, C)]], u_v.at[b], sems.at[b])
            pltpu.async_copy(
                pview.at[dsts.at[pl.ds(issue_ci * C, C)]], v_v.at[b], sems.at[b])
        return carry

    lax.fori_loop(0, NCHUNK, chunk_body, 0)

    for b in range(NB):
        pltpu.make_async_copy(
            o_v.at[b], out_hbm.at[pl.ds(0, C)], osems.at[b]).wait()


@jax.jit
def kernel(x, edge_index):
    mesh = plsc.VectorSubcoreMesh(
        core_axis_name="c", subcore_axis_name="s",
        num_cores=N_CORES, num_subcores=N_SUBCORES)
    f = pl.kernel(
        _dot_body,
        out_type=[
            jax.ShapeDtypeStruct((E,), jnp.float32),
            jax.ShapeDtypeStruct((N_CORES * N, W), jnp.int32),
        ],
        mesh=mesh,
        compiler_params=pltpu.CompilerParams(needs_layout_passes=False, use_tc_tiling_on_sc=False),
        scratch_types=[
            pltpu.VMEM((EPW,), jnp.int32),
            pltpu.VMEM((EPW,), jnp.int32),
            pltpu.VMEM((PNB, PR, D), jnp.float32),
            pltpu.VMEM((PNB, PR, W), jnp.int32),
            pltpu.VMEM((NB, C, W), jnp.int32),
            pltpu.VMEM((NB, C, W), jnp.int32),
            pltpu.VMEM((NB, C), jnp.float32),
            pltpu.SemaphoreType.DMA((NB,)),
            pltpu.SemaphoreType.DMA((NB,)),
            pltpu.SemaphoreType.DMA((PNB,)),
            pltpu.SemaphoreType.DMA((PNB,)),
            pltpu.SemaphoreType.DMA((2,)),
        ],
    )
    scores, _ = f(x, edge_index)
    return scores


# speedup vs baseline: 17.0035x; 1.0063x over previous
import jax
import jax.numpy as jnp
from jax import lax
from jax.experimental import pallas as pl
from jax.experimental.pallas import tpu as pltpu
from jax.experimental.pallas import tpu_sc as plsc

E = 320000
N = 10000
D = 128
W = D // 2
N_CORES = 2
N_SUBCORES = 16
NW = N_CORES * N_SUBCORES
EPW = E // NW
C = 80
NCHUNK = EPW // C
G = C // 16
KW = W // 16
NB = 5
RPT = N // N_SUBCORES
PR = 25
NP1 = RPT // PR
PNB = 3


def _dot_body(x_hbm, ei_hbm, out_hbm, packed_hbm,
              srcs, dsts, xf_v, pk_v, u_v, v_v, o_v,
              sems, osems, pisems, posems, isems):
    cid = lax.axis_index("c")
    sid = lax.axis_index("s")
    wid = sid * N_CORES + cid
    iota = lax.iota(jnp.int32, 16)
    ebase = wid * EPW
    row0 = sid * RPT
    pbase = cid * N

    pltpu.async_copy(ei_hbm.at[0, pl.ds(ebase, EPW)], srcs, isems.at[0])
    pltpu.async_copy(ei_hbm.at[1, pl.ds(ebase, EPW)], dsts, isems.at[1])

    def p1_issue(i, b):
        pltpu.async_copy(
            x_hbm.at[pl.ds(row0 + i * PR, PR)], xf_v.at[b], pisems.at[b])

    for b in range(PNB):
        p1_issue(b, b)

    def p1_body(i, carry):
        b = lax.rem(i, PNB)

        @pl.when(i >= PNB)
        def _():
            pltpu.make_async_copy(
                pk_v.at[b], packed_hbm.at[pl.ds(0, PR)], posems.at[b]).wait()

        pltpu.make_async_copy(
            x_hbm.at[pl.ds(0, PR)], xf_v.at[b], pisems.at[b]).wait()

        for r in range(PR):
            for k in range(KW):
                a = xf_v[b, r, pl.ds(k * 32, 16)]
                c2 = xf_v[b, r, pl.ds(k * 32 + 16, 16)]
                w = plsc.pack(a, c2, format=plsc.PackFormat.INTERLEAVED)
                pk_v[b, r, pl.ds(k * 16, 16)] = plsc.bitcast(w, jnp.int32)

        pltpu.async_copy(
            pk_v.at[b],
            packed_hbm.at[pl.ds(pbase + row0 + i * PR, PR)], posems.at[b])

        @pl.when(i + PNB < NP1)
        def _():
            p1_issue(i + PNB, b)
        return carry

    lax.fori_loop(0, NP1, p1_body, 0)

    for b in range(PNB):
        pltpu.make_async_copy(
            pk_v.at[b], packed_hbm.at[pl.ds(0, PR)], posems.at[b]).wait()

    plsc.subcore_barrier()

    pview = packed_hbm.at[pl.ds(pbase, N)]

    pltpu.make_async_copy(ei_hbm.at[0, pl.ds(ebase, EPW)], srcs, isems.at[0]).wait()
    pltpu.make_async_copy(ei_hbm.at[1, pl.ds(ebase, EPW)], dsts, isems.at[1]).wait()

    def issue(ci, b):
        pltpu.async_copy(pview.at[srcs.at[pl.ds(ci * C, C)]], u_v.at[b], sems.at[b])
        pltpu.async_copy(pview.at[dsts.at[pl.ds(ci * C, C)]], v_v.at[b], sems.at[b])

    for b in range(NB):
        issue(b, b)

    def chunk_body(ci, carry):
        b = lax.rem(ci, NB)

        @pl.when(ci >= NB)
        def _():
            pltpu.make_async_copy(
                o_v.at[b], out_hbm.at[pl.ds(0, C)], osems.at[b]).wait()

        pltpu.make_async_copy(
            pview.at[srcs.at[pl.ds(ci * C, C)]], u_v.at[b], sems.at[b]).wait()
        pltpu.make_async_copy(
            pview.at[dsts.at[pl.ds(ci * C, C)]], v_v.at[b], sems.at[b]).wait()

        for g in range(G):
            sv = jnp.zeros((16,), jnp.float32)
            for j in range(16):
                e = g * 16 + j
                acc = None
                for k in range(KW):
                    wu = u_v[b, e, pl.ds(k * 16, 16)]
                    wv = v_v[b, e, pl.ds(k * 16, 16)]
                    bu = plsc.bitcast(wu, jnp.bfloat16)
                    bv = plsc.bitcast(wv, jnp.bfloat16)
                    pr = bu * bv
                    p0, p1 = plsc.unpack(pr, format=plsc.PackFormat.INTERLEAVED)
                    t = p0 + p1
                    acc = t if acc is None else acc + t
                s = jnp.sum(acc)
                sv = jnp.where(iota == j, s, sv)
            o_v[b, pl.ds(g * 16, 16)] = sv

        pltpu.async_copy(o_v.at[b], out_hbm.at[pl.ds(ebase + ci * C, C)], osems.at[b])

        @pl.when(ci + NB < NCHUNK)
        def _():
            issue_ci = ci + NB
            pltpu.async_copy(
                pview.at[srcs.at[pl.ds(issue_ci * C, C)]], u_v.at[b], sems.at[b])
            pltpu.async_copy(
                pview.at[dsts.at[pl.ds(issue_ci * C, C)]], v_v.at[b], sems.at[b])
        return carry

    lax.fori_loop(0, NCHUNK, chunk_body, 0)

    for b in range(NB):
        pltpu.make_async_copy(
            o_v.at[b], out_hbm.at[pl.ds(0, C)], osems.at[b]).wait()


@jax.jit
def kernel(x, edge_index):
    mesh = plsc.VectorSubcoreMesh(
        core_axis_name="c", subcore_axis_name="s",
        num_cores=N_CORES, num_subcores=N_SUBCORES)
    f = pl.kernel(
        _dot_body,
        out_type=[
            jax.ShapeDtypeStruct((E,), jnp.float32),
            jax.ShapeDtypeStruct((N_CORES * N, W), jnp.int32),
        ],
        mesh=mesh,
        compiler_params=pltpu.CompilerParams(needs_layout_passes=False, use_tc_tiling_on_sc=False),
        scratch_types=[
            pltpu.VMEM((EPW,), jnp.int32),
            pltpu.VMEM((EPW,), jnp.int32),
            pltpu.VMEM((PNB, PR, D), jnp.float32),
            pltpu.VMEM((PNB, PR, W), jnp.int32),
            pltpu.VMEM((NB, C, W), jnp.int32),
            pltpu.VMEM((NB, C, W), jnp.int32),
            pltpu.VMEM((NB, C), jnp.float32),
            pltpu.SemaphoreType.DMA((NB,)),
            pltpu.SemaphoreType.DMA((NB,)),
            pltpu.SemaphoreType.DMA((PNB,)),
            pltpu.SemaphoreType.DMA((PNB,)),
            pltpu.SemaphoreType.DMA((2,)),
        ],
    )
    scores, _ = f(x, edge_index)
    return scores
